# baseline (speedup 1.0000x reference)
import functools

import jax
import jax.numpy as jnp
from jax import lax
from jax.experimental import pallas as pl
from jax.experimental.pallas import tpu as pltpu
from jax.experimental.pallas import tpu_sc as plsc

BATCH = 16384
NC, NS = 2, 16
NW = NC * NS
BPW = BATCH // NW
LANES = 16
OUT_D = 28
CHAR_D = 6
BAND_D = 20
LOC_D, FACT_D, GOAL_D = 4, 6, 4
COMB = 4 * BPW
TBL_R = 44


def _sc_encode(comb_idx, achar, comb_tab, c0, c1, c2, c3, c4, c5):
    mesh = plsc.VectorSubcoreMesh(core_axis_name="c", subcore_axis_name="s")

    @functools.partial(
        pl.kernel,
        mesh=mesh,
        compiler_params=pltpu.CompilerParams(
            use_tc_tiling_on_sc=False,
            needs_layout_passes=False),
        out_type=jax.ShapeDtypeStruct((BATCH, OUT_D), jnp.float32),
        scratch_types=[
            pltpu.VMEM((COMB,), jnp.int32),
            pltpu.VMEM((BPW,), jnp.int32),
            pltpu.VMEM((TBL_R, 8), jnp.float32),
            pltpu.VMEM((BPW, 8), jnp.float32),
            pltpu.VMEM((CHAR_D * BPW,), jnp.float32),
            pltpu.VMEM((BPW, BAND_D), jnp.float32),
            pltpu.SemaphoreType.DMA,
            pltpu.SemaphoreType.DMA,
        ],
    )
    def k(ci_h, ca_h, tab_h, c0_h, c1_h, c2_h, c3_h, c4_h, c5_h, out_h,
          ci_v, ca_v, tbl_v, rt_v, rcc_v, band_v, sem_a, sem_b):
        wid = lax.axis_index("s") * NC + lax.axis_index("c")
        base = wid * BPW
        pltpu.sync_copy(ca_h.at[wid], ca_v)
        g_char = []
        for c, col_h in enumerate((c0_h, c1_h, c2_h, c3_h, c4_h, c5_h)):
            g_char.append(pltpu.async_copy(
                col_h.at[ca_v], rcc_v.at[pl.ds(c * BPW, BPW)], sem_b))
        pltpu.sync_copy(ci_h.at[wid], ci_v)
        pltpu.sync_copy(tab_h, tbl_v)

        iota = lax.iota(jnp.int32, LANES)

        c8 = iota % 8
        r8 = iota // 8

        def type_body(i, r):
            rowid = plsc.load_gather(ci_v, [r])
            v = plsc.load_gather(tbl_v, [rowid, c8])
            plsc.store_scatter(rt_v, [r, c8], v)
            return r + 2

        lax.fori_loop(0, BPW * 8 // LANES, type_body, r8, unroll=8)
        out_t = pltpu.async_copy(
            rt_v, out_h.at[pl.ds(base, BPW), pl.ds(0, 8)], sem_a)

        cs4 = iota % 4
        r4 = iota // 4

        def loc_body(i, r):
            rowid = plsc.load_gather(ci_v, [r + BPW]) + 8
            v = plsc.load_gather(tbl_v, [rowid, cs4])
            plsc.store_scatter(band_v, [r, cs4 + CHAR_D], v)
            return r + 4

        fr, fcs = [], []
        for p in range(3):
            e = iota + (p * LANES)
            fr.append(e // FACT_D)
            fcs.append(e % FACT_D)

        def fact_body(i, r):
            for p in range(3):
                rp = r + fr[p]
                rowid = plsc.load_gather(ci_v, [rp + 2 * BPW]) + 12
                v = plsc.load_gather(tbl_v, [rowid, fcs[p]])
                plsc.store_scatter(band_v, [rp, fcs[p] + CHAR_D + LOC_D], v)
            return r + 8

        def goal_body(i, r):
            rowid = plsc.load_gather(ci_v, [r + 3 * BPW]) + 33
            v = plsc.load_gather(tbl_v, [rowid, cs4])
            plsc.store_scatter(band_v, [r, cs4 + (BAND_D - GOAL_D)], v)
            return r + 4

        lax.fori_loop(0, BPW * LOC_D // LANES, loc_body, r4, unroll=4)
        lax.fori_loop(0, BPW * FACT_D // (3 * LANES), fact_body,
                      iota * 0, unroll=2)
        lax.fori_loop(0, BPW * GOAL_D // LANES, goal_body, r4, unroll=4)

        for c in range(CHAR_D):
            g_char[c].wait()

            def char_body(i, r, c=c):
                v = rcc_v[pl.ds(c * BPW + i * LANES, LANES)]
                plsc.store_scatter(band_v, [r, iota * 0 + c], v)
                return r + LANES

            lax.fori_loop(0, BPW // LANES, char_body, iota, unroll=4)

        pltpu.sync_copy(band_v, out_h.at[pl.ds(base, BPW), pl.ds(8, BAND_D)])
        out_t.wait()

    return k(comb_idx, achar, comb_tab, c0, c1, c2, c3, c4, c5)


def kernel(atype, achar, aloc, afact, agoal,
           type_emb, char_emb, loc_emb, fact_emb, goal_emb):
    def pad8(t):
        return jnp.pad(t, ((0, 0), (0, 8 - t.shape[1])))

    comb_tab = jnp.concatenate(
        [type_emb, pad8(loc_emb), pad8(fact_emb), pad8(goal_emb)])

    def w(a):
        return a.astype(jnp.int32).reshape(NW, BPW)

    comb_idx = jnp.concatenate(
        [w(atype), w(aloc), w(afact), w(agoal)], axis=1)

    charT = char_emb.T
    cols = [lax.optimization_barrier(charT[c]) for c in range(CHAR_D)]

    return _sc_encode(comb_idx, w(achar), comb_tab, *cols)


# speedup vs baseline: 1.0026x; 1.0026x over previous
import functools

import jax
import jax.numpy as jnp
from jax import lax
from jax.experimental import pallas as pl
from jax.experimental.pallas import tpu as pltpu
from jax.experimental.pallas import tpu_sc as plsc

BATCH = 16384
NC, NS = 2, 16
NW = NC * NS
BPW = BATCH // NW
LANES = 16
OUT_D = 28
CHAR_D = 6
BAND_D = 20
LOC_D, FACT_D, GOAL_D = 4, 6, 4
COMB = 4 * BPW
TBL_R = 44


def _sc_encode(comb_idx, achar, comb_tab, c0, c1, c2, c3, c4, c5):
    mesh = plsc.VectorSubcoreMesh(core_axis_name="c", subcore_axis_name="s")

    @functools.partial(
        pl.kernel,
        mesh=mesh,
        compiler_params=pltpu.CompilerParams(
            use_tc_tiling_on_sc=False,
            needs_layout_passes=False),
        out_type=jax.ShapeDtypeStruct((BATCH, OUT_D), jnp.float32),
        scratch_types=[
            pltpu.VMEM((COMB,), jnp.int32),
            pltpu.VMEM((BPW,), jnp.int32),
            pltpu.VMEM((TBL_R, 8), jnp.float32),
            pltpu.VMEM((BPW, 8), jnp.float32),
            pltpu.VMEM((CHAR_D * BPW,), jnp.float32),
            pltpu.VMEM((BPW, BAND_D), jnp.float32),
            pltpu.SemaphoreType.DMA,
            pltpu.SemaphoreType.DMA,
        ],
    )
    def k(ci_h, ca_h, tab_h, c0_h, c1_h, c2_h, c3_h, c4_h, c5_h, out_h,
          ci_v, ca_v, tbl_v, rt_v, rcc_v, band_v, sem_a, sem_b):
        wid = lax.axis_index("s") * NC + lax.axis_index("c")
        base = wid * BPW
        pltpu.sync_copy(ca_h.at[wid], ca_v)
        g_char = []
        for c, col_h in enumerate((c0_h, c1_h, c2_h, c3_h, c4_h, c5_h)):
            g_char.append(pltpu.async_copy(
                col_h.at[ca_v], rcc_v.at[pl.ds(c * BPW, BPW)], sem_b))
        pltpu.sync_copy(ci_h.at[wid], ci_v)
        pltpu.sync_copy(tab_h, tbl_v)

        iota = lax.iota(jnp.int32, LANES)

        c8 = iota % 8
        r8 = iota // 8

        def type_body(i, r):
            rowid = plsc.load_gather(ci_v, [r])
            v = plsc.load_gather(tbl_v, [rowid, c8])
            plsc.store_scatter(rt_v, [r, c8], v)
            return r + 2

        lax.fori_loop(0, BPW * 8 // LANES, type_body, r8, unroll=8)
        out_t = pltpu.async_copy(
            rt_v, out_h.at[pl.ds(base, BPW), pl.ds(0, 8)], sem_a)

        cs4 = iota % 4
        r4 = iota // 4

        def loc_body(i, r):
            rowid = plsc.load_gather(ci_v, [r + BPW]) + 8
            v = plsc.load_gather(tbl_v, [rowid, cs4])
            plsc.store_scatter(band_v, [r, cs4 + CHAR_D], v)
            return r + 4

        fr, fcs = [], []
        for p in range(3):
            e = iota + (p * LANES)
            fr.append(e // FACT_D)
            fcs.append(e % FACT_D)

        def fact_body(i, r):
            for p in range(3):
                rp = r + fr[p]
                rowid = plsc.load_gather(ci_v, [rp + 2 * BPW]) + 12
                v = plsc.load_gather(tbl_v, [rowid, fcs[p]])
                plsc.store_scatter(band_v, [rp, fcs[p] + CHAR_D + LOC_D], v)
            return r + 8

        def goal_body(i, r):
            rowid = plsc.load_gather(ci_v, [r + 3 * BPW]) + 33
            v = plsc.load_gather(tbl_v, [rowid, cs4])
            plsc.store_scatter(band_v, [r, cs4 + (BAND_D - GOAL_D)], v)
            return r + 4

        lax.fori_loop(0, BPW * LOC_D // LANES, loc_body, r4, unroll=4)
        lax.fori_loop(0, BPW * FACT_D // (3 * LANES), fact_body,
                      iota * 0, unroll=2)
        lax.fori_loop(0, BPW * GOAL_D // LANES, goal_body, r4, unroll=4)

        for c in range(CHAR_D):
            g_char[c].wait()

            def char_body(i, r, c=c):
                v = rcc_v[pl.ds(c * BPW + i * LANES, LANES)]
                plsc.store_scatter(band_v, [r, iota * 0 + c], v)
                return r + LANES

            lax.fori_loop(0, BPW // LANES, char_body, iota, unroll=4)

        pltpu.sync_copy(band_v, out_h.at[pl.ds(base, BPW), pl.ds(8, BAND_D)])
        out_t.wait()

    return k(comb_idx, achar, comb_tab, c0, c1, c2, c3, c4, c5)


def kernel(atype, achar, aloc, afact, agoal,
           type_emb, char_emb, loc_emb, fact_emb, goal_emb):
    def pad8(t):
        return jnp.pad(t, ((0, 0), (0, 8 - t.shape[1])))

    comb_tab = jnp.concatenate(
        [type_emb, pad8(loc_emb), pad8(fact_emb), pad8(goal_emb)])

    def w(a):
        return a.astype(jnp.int32).reshape(NW, BPW)

    comb_idx = jnp.concatenate(
        [w(atype), w(aloc), w(afact), w(agoal)], axis=1)

    charT = char_emb.T
    cols = [charT[c] for c in range(CHAR_D)]

    return _sc_encode(comb_idx, w(achar), comb_tab, *cols)
